# baseline (speedup 1.0000x reference)
import functools

import jax
import jax.numpy as jnp
from jax import lax
from jax.experimental import pallas as pl
from jax.experimental.pallas import tpu as pltpu
from jax.experimental.pallas import tpu_sc as plsc

N_NODES = 10000
N_EDGES = 320000
D_EMB = 64
HEADS = 4
EPS = 1e-6

NEP = 327680
VW = 128
NROWS_HALF = 5120
NROWS_CORE = 5248
SLOW_CORE = 1
N_SLOW_CHUNKS = 48


def _dot(a, b):
    return jnp.dot(a, b)


def _ln(x, g, b):
    mu = jnp.mean(x, axis=-1, keepdims=True)
    v = jnp.mean((x - mu) * (x - mu), axis=-1, keepdims=True)
    return (x - mu) * jax.lax.rsqrt(v + EPS) * g + b


def _mlp2_body(x, w1, b1, w2, b2, o):
    h = jnp.maximum(_dot(x[...], w1[...]) + b1[...], 0.0)
    o[...] = _dot(h, w2[...]) + b2[...]


def _mlp2(x, w1, b1, w2, b2, blk):
    n = x.shape[0]
    return pl.pallas_call(
        _mlp2_body,
        grid=(n // blk,),
        in_specs=[
            pl.BlockSpec((blk, x.shape[1]), lambda i: (i, 0)),
            pl.BlockSpec(w1.shape, lambda i: (0, 0)),
            pl.BlockSpec(b1.shape, lambda i: (0,)),
            pl.BlockSpec(w2.shape, lambda i: (0, 0)),
            pl.BlockSpec(b2.shape, lambda i: (0,)),
        ],
        out_specs=pl.BlockSpec((blk, w2.shape[1]), lambda i: (i, 0)),
        out_shape=jax.ShapeDtypeStruct((n, w2.shape[1]), jnp.float32),
    )(x, w1, b1, w2, b2)


def _mlp2p_body(x, w1, b1, w2, b2, wp, o, op):
    h = jnp.maximum(_dot(x[...], w1[...]) + b1[...], 0.0)
    e = _dot(h, w2[...]) + b2[...]
    o[...] = e
    op[...] = _dot(e, wp[...])


def _mlp2p(x, w1, b1, w2, b2, wp, blk):
    n = x.shape[0]
    return pl.pallas_call(
        _mlp2p_body,
        grid=(n // blk,),
        in_specs=[
            pl.BlockSpec((blk, x.shape[1]), lambda i: (i, 0)),
            pl.BlockSpec(w1.shape, lambda i: (0, 0)),
            pl.BlockSpec(b1.shape, lambda i: (0,)),
            pl.BlockSpec(w2.shape, lambda i: (0, 0)),
            pl.BlockSpec(b2.shape, lambda i: (0,)),
            pl.BlockSpec(wp.shape, lambda i: (0, 0)),
        ],
        out_specs=[
            pl.BlockSpec((blk, w2.shape[1]), lambda i: (i, 0)),
            pl.BlockSpec((blk, wp.shape[1]), lambda i: (i, 0)),
        ],
        out_shape=[
            jax.ShapeDtypeStruct((n, w2.shape[1]), jnp.float32),
            jax.ShapeDtypeStruct((n, wp.shape[1]), jnp.float32),
        ],
    )(x, w1, b1, w2, b2, wp)


def _edge_update_body(swi, ndp, e0, b1, w2, b2, eg, eb, w1b, wq, wk,
                      o_e1, o_vals):
    h = jnp.maximum(swi[...] + _dot(e0[...], w1b[...]) + b1[...], 0.0)
    upd = _dot(h, w2[...]) + b2[...]
    e1 = _ln(upd + e0[...], eg[...], eb[...])
    o_e1[...] = e1
    kk = _dot(e1, wk[...])
    qq = _dot(ndp[:, :D_EMB], wq[...])
    prod = kk * qq
    idx = lax.broadcasted_iota(jnp.int32, (1, D_EMB), 1)
    e1m = jnp.where(idx < D_EMB - 1, e1, 1.0)
    for c in range(2):
        exa = jnp.exp(jnp.sum(prod[:, (2 * c) * D_EMB:(2 * c + 1) * D_EMB],
                              axis=1, keepdims=True))
        exb = jnp.exp(jnp.sum(prod[:, (2 * c + 1) * D_EMB:(2 * c + 2) * D_EMB],
                              axis=1, keepdims=True))
        o_vals[c] = jnp.concatenate([e1m * exa, e1m * exb], axis=1)


def _edge_update(swi, ndp, e0, b1, w2, b2, eg, eb, w1b, wq, wk):
    blk = 2048
    return pl.pallas_call(
        _edge_update_body,
        grid=(NEP // blk,),
        in_specs=[
            pl.BlockSpec((blk, 2 * D_EMB), lambda i: (i, 0)),
            pl.BlockSpec((blk, 2 * D_EMB), lambda i: (i, 0)),
            pl.BlockSpec((blk, D_EMB), lambda i: (i, 0)),
            pl.BlockSpec(b1.shape, lambda i: (0,)),
            pl.BlockSpec(w2.shape, lambda i: (0, 0)),
            pl.BlockSpec(b2.shape, lambda i: (0,)),
            pl.BlockSpec(eg.shape, lambda i: (0,)),
            pl.BlockSpec(eb.shape, lambda i: (0,)),
            pl.BlockSpec(w1b.shape, lambda i: (0, 0)),
            pl.BlockSpec(wq.shape, lambda i: (0, 0)),
            pl.BlockSpec(wk.shape, lambda i: (0, 0)),
        ],
        out_specs=[
            pl.BlockSpec((blk, D_EMB), lambda i: (i, 0)),
            pl.BlockSpec((2, blk, VW), lambda i: (0, i, 0)),
        ],
        out_shape=[
            jax.ShapeDtypeStruct((NEP, D_EMB), jnp.float32),
            jax.ShapeDtypeStruct((2, NEP, VW), jnp.float32),
        ],
    )(swi, ndp, e0, b1, w2, b2, eg, eb, w1b, wq, wk)


def _node_update_body(num, ne, eg, eb, w1a, w1b, b1, w2, b2, ng, nb, wp, o, op):
    idx = lax.broadcasted_iota(jnp.int32, (1, D_EMB), 1)
    egv, ebv = eg[...], eb[...]
    last = D_EMB - 1
    inv = jnp.where(idx[0] < last, 1.0 / egv, 0.0)
    c2 = jnp.sum(ebv * inv)
    eg63 = jnp.sum(jnp.where(idx[0] == last, egv, 0.0))
    eb63 = jnp.sum(jnp.where(idx[0] == last, ebv, 0.0))
    heads = []
    for h in range(HEADS):
        blk = num[:, h * D_EMB:(h + 1) * D_EMB]
        den = jnp.sum(jnp.where(idx == last, blk, 0.0), axis=1, keepdims=True)
        s1 = jnp.sum(blk * inv, axis=1, keepdims=True)
        n63 = eb63 * den - eg63 * (s1 - den * c2)
        nf = jnp.where(idx < last, blk, n63)
        heads.append(nf / (den + 1e-9))
    attn = jnp.concatenate(heads, axis=1)
    hh = jnp.maximum(_dot(attn, w1a[...]) + _dot(ne[...], w1b[...])
                     + b1[...], 0.0)
    upd = _dot(hh, w2[...]) + b2[...]
    n2 = _ln(upd + ne[...], ng[...], nb[...])
    o[...] = n2
    op[...] = _dot(n2, wp[...])


def _node_update(num, ne, eg, eb, w1a, w1b, b1, w2, b2, ng, nb, wp):
    blk = 2000
    return pl.pallas_call(
        _node_update_body,
        grid=(N_NODES // blk,),
        in_specs=[
            pl.BlockSpec((blk, HEADS * D_EMB), lambda i: (i, 0)),
            pl.BlockSpec((blk, D_EMB), lambda i: (i, 0)),
            pl.BlockSpec(eg.shape, lambda i: (0,)),
            pl.BlockSpec(eb.shape, lambda i: (0,)),
            pl.BlockSpec(w1a.shape, lambda i: (0, 0)),
            pl.BlockSpec(w1b.shape, lambda i: (0, 0)),
            pl.BlockSpec(b1.shape, lambda i: (0,)),
            pl.BlockSpec(w2.shape, lambda i: (0, 0)),
            pl.BlockSpec(b2.shape, lambda i: (0,)),
            pl.BlockSpec(ng.shape, lambda i: (0,)),
            pl.BlockSpec(nb.shape, lambda i: (0,)),
            pl.BlockSpec(wp.shape, lambda i: (0, 0)),
        ],
        out_specs=[
            pl.BlockSpec((blk, D_EMB), lambda i: (i, 0)),
            pl.BlockSpec((blk, wp.shape[1]), lambda i: (i, 0)),
        ],
        out_shape=[
            jax.ShapeDtypeStruct((N_NODES, D_EMB), jnp.float32),
            jax.ShapeDtypeStruct((N_NODES, wp.shape[1]), jnp.float32),
        ],
    )(num, ne, eg, eb, w1a, w1b, b1, w2, b2, ng, nb, wp)


def _out_head_body(us, ud, e1, w1b, b1, w2, b2, o):
    h = jnp.maximum(us[...] + ud[...] + _dot(e1[...], w1b[...]) + b1[...], 0.0)
    o[...] = _dot(h, w2[...]) + b2[...]


def _out_head(us, ud, e1, w1b, b1, w2, b2):
    blk = 2048
    return pl.pallas_call(
        _out_head_body,
        grid=(NEP // blk,),
        in_specs=[
            pl.BlockSpec((blk, 2 * D_EMB), lambda i: (i, 0)),
            pl.BlockSpec((blk, 2 * D_EMB), lambda i: (i, 0)),
            pl.BlockSpec((blk, D_EMB), lambda i: (i, 0)),
            pl.BlockSpec(w1b.shape, lambda i: (0, 0)),
            pl.BlockSpec(b1.shape, lambda i: (0,)),
            pl.BlockSpec(w2.shape, lambda i: (0, 0)),
            pl.BlockSpec(b2.shape, lambda i: (0,)),
        ],
        out_specs=pl.BlockSpec((blk, 1), lambda i: (i, 0)),
        out_shape=jax.ShapeDtypeStruct((NEP, 1), jnp.float32),
    )(us, ud, e1, w1b, b1, w2, b2)


_SC_MESH = dict(core_axis_name="c", subcore_axis_name="s")


def _sc_gather2(ta, tb, ia, ib):
    CH = 128
    PER_W = NEP // 32
    NCH = PER_W // CH
    SUP = 8
    NSUP = NCH // SUP
    W = ta.shape[1]

    @functools.partial(
        pl.kernel, mesh=plsc.VectorSubcoreMesh(**_SC_MESH),
        out_type=[jax.ShapeDtypeStruct((NEP, W), jnp.float32),
                  jax.ShapeDtypeStruct((NEP, W), jnp.float32)],
        scratch_types=[
            pltpu.VMEM((SUP * CH,), jnp.int32),
            pltpu.VMEM((SUP * CH,), jnp.int32),
            pltpu.VMEM((3, CH, W), jnp.float32),
            pltpu.VMEM((3, CH, W), jnp.float32),
            pltpu.SemaphoreType.DMA((3,)),
            pltpu.SemaphoreType.DMA((3,)),
            pltpu.SemaphoreType.DMA((3,)),
            pltpu.SemaphoreType.DMA((3,)),
        ],
    )
    def k(ta_h, tb_h, ia_h, ib_h, oa_h, ob_h, iav, ibv, ra, rb,
          sga, sgb, swa, swb):
        c = lax.axis_index("c")
        s = lax.axis_index("s")
        n_slow = N_SLOW_CHUNKS
        n_fast = 2 * NCH - n_slow
        nsup_me = jnp.where(c == SLOW_CORE, n_slow // SUP, n_fast // SUP)
        base0 = jnp.where(c == SLOW_CORE, s * n_slow,
                          16 * n_slow + s * n_fast) * CH
        NB = 3

        def sup_body(g, carry):
            sbase = base0 + g * (SUP * CH)
            pltpu.sync_copy(ia_h.at[pl.ds(sbase, SUP * CH)], iav)
            pltpu.sync_copy(ib_h.at[pl.ds(sbase, SUP * CH)], ibv)
            for b in range(SUP):
                sl = b % NB
                if b >= NB:
                    pltpu.make_async_copy(
                        ra.at[sl], oa_h.at[pl.ds(sbase, CH)], swa.at[sl]).wait()
                    pltpu.make_async_copy(
                        rb.at[sl], ob_h.at[pl.ds(sbase, CH)], swb.at[sl]).wait()
                pltpu.async_copy(ta_h.at[iav.at[pl.ds(b * CH, CH)]],
                                 ra.at[sl], sga.at[sl])
                pltpu.async_copy(tb_h.at[ibv.at[pl.ds(b * CH, CH)]],
                                 rb.at[sl], sgb.at[sl])
                if b >= 1:
                    psl = (b - 1) % NB
                    pbase = sbase + (b - 1) * CH
                    pltpu.make_async_copy(
                        ta_h.at[iav.at[pl.ds(0, CH)]], ra.at[psl],
                        sga.at[psl]).wait()
                    pltpu.async_copy(ra.at[psl], oa_h.at[pl.ds(pbase, CH)],
                                     swa.at[psl])
                    pltpu.make_async_copy(
                        tb_h.at[ibv.at[pl.ds(0, CH)]], rb.at[psl],
                        sgb.at[psl]).wait()
                    pltpu.async_copy(rb.at[psl], ob_h.at[pl.ds(pbase, CH)],
                                     swb.at[psl])
            lsl = (SUP - 1) % NB
            lbase = sbase + (SUP - 1) * CH
            pltpu.make_async_copy(
                ta_h.at[iav.at[pl.ds(0, CH)]], ra.at[lsl], sga.at[lsl]).wait()
            pltpu.async_copy(ra.at[lsl], oa_h.at[pl.ds(lbase, CH)], swa.at[lsl])
            pltpu.make_async_copy(
                tb_h.at[ibv.at[pl.ds(0, CH)]], rb.at[lsl], sgb.at[lsl]).wait()
            pltpu.async_copy(rb.at[lsl], ob_h.at[pl.ds(lbase, CH)], swb.at[lsl])
            for sl in range(NB):
                pltpu.make_async_copy(
                    ra.at[sl], oa_h.at[pl.ds(sbase, CH)], swa.at[sl]).wait()
                pltpu.make_async_copy(
                    rb.at[sl], ob_h.at[pl.ds(sbase, CH)], swb.at[sl]).wait()
            return carry

        lax.fori_loop(0, nsup_me, sup_body, 0)

    return k(ta, tb, ia, ib)


def _sc_scatter(vals, dstsc, zn):
    CH = 128
    PER_T = NEP // 16
    NCH = PER_T // CH
    SUP = 8
    NSUP = NCH // SUP
    RPT = NROWS_CORE // 16

    @functools.partial(
        pl.kernel, mesh=plsc.VectorSubcoreMesh(**_SC_MESH),
        out_type=jax.ShapeDtypeStruct((2, 2, NROWS_CORE, VW), jnp.float32),
        scratch_types=[
            pltpu.VMEM((SUP, CH), jnp.int32),
            pltpu.VMEM((2, CH, VW), jnp.float32),
            pltpu.VMEM((RPT, VW), jnp.float32),
            pltpu.VMEM_SHARED((NROWS_CORE, VW), jnp.float32),
            pltpu.SemaphoreType.DMA((2,)),
            pltpu.SemaphoreType.DMA((2,)),
        ],
    )
    def k(vals_h, dsts_h, zn_h, on_h, iv, vv, ov, tab, sv, ss):
        c = lax.axis_index("c")
        s = lax.axis_index("s")
        for p in range(2):
            pltpu.sync_copy(zn_h, tab.at[pl.ds(s * RPT, RPT)])
            plsc.subcore_barrier()

            def sup_body(g, carry):
                crow = (s * PER_T + g * (SUP * CH)) // CH
                pltpu.sync_copy(
                    dsts_h.at[c, pl.ds(pl.multiple_of(crow, 8), SUP)], iv)
                for b in range(SUP):
                    sl = b % 2
                    if b >= 2:
                        pltpu.make_async_copy(
                            vv.at[sl], tab.at[iv.at[0]], ss.at[sl]).wait()
                    pltpu.async_copy(
                        vals_h.at[p, pl.ds((crow + b) * CH, CH)],
                        vv.at[sl], sv.at[sl])
                    if b >= 1:
                        psl = (b - 1) % 2
                        pltpu.make_async_copy(
                            vals_h.at[p, pl.ds(crow * CH, CH)],
                            vv.at[psl], sv.at[psl]).wait()
                        pltpu.async_copy(vv.at[psl], tab.at[iv.at[b - 1]],
                                         ss.at[psl], add=True)
                lsl = (SUP - 1) % 2
                pltpu.make_async_copy(
                    vals_h.at[p, pl.ds(crow * CH, CH)],
                    vv.at[lsl], sv.at[lsl]).wait()
                pltpu.async_copy(vv.at[lsl], tab.at[iv.at[SUP - 1]],
                                 ss.at[lsl], add=True)
                for sl in range(2):
                    pltpu.make_async_copy(
                        vv.at[sl], tab.at[iv.at[0]], ss.at[sl]).wait()
                return carry

            lax.fori_loop(0, NSUP, sup_body, 0)
            plsc.subcore_barrier()
            pltpu.sync_copy(tab.at[pl.ds(s * RPT, RPT)], ov)
            pltpu.sync_copy(ov, on_h.at[p, c, pl.ds(s * RPT, RPT)])

    return k(vals, dstsc, zn)


def kernel(node_feats, edge_feats, edge_index, Wn1, bn1, Wn2, bn2, We1, be1, We2, be2,
           Wne1, bne1, Wne2, bne2, Wq, Wk, Wen1, ben1, Wen2, ben2,
           eg, eb, ng, nb, Wo1, bo1, Wo2, bo2):
    src = edge_index[0]
    dst = edge_index[1]
    pad = NEP - N_EDGES
    srcp = jnp.concatenate([src, jnp.zeros((pad,), jnp.int32)])
    dstp = jnp.concatenate([dst, jnp.zeros((pad,), jnp.int32)])
    dsts = jnp.concatenate(
        [dst, N_NODES + (jnp.arange(pad, dtype=jnp.int32) % (2 * NROWS_HALF - N_NODES))])
    dump = NROWS_HALF + (dsts & 127)
    dstsc = jnp.stack([
        jnp.where(dsts < NROWS_HALF, dsts, dump),
        jnp.where(dsts >= NROWS_HALF, dsts - NROWS_HALF, dump),
    ]).reshape(2, NEP // 128, 128)

    scale = 1.0 / jnp.sqrt(jnp.asarray(D_EMB, dtype=jnp.float32))
    wq_cat = Wq.transpose(1, 0, 2).reshape(D_EMB, HEADS * D_EMB) * scale
    wk_cat = Wk.transpose(1, 0, 2).reshape(D_EMB, HEADS * D_EMB)

    node_emb, t_src = _mlp2p(node_feats, Wn1, bn1, Wn2, bn2, Wne1[:D_EMB], 2000)
    t_dst = jnp.pad(node_emb, ((0, 0), (0, D_EMB)))
    efp = jnp.concatenate([edge_feats,
                           jnp.zeros((pad, edge_feats.shape[1]), jnp.float32)])
    e0 = _mlp2(efp, We1, be1, We2, be2, 2048)

    swi, ndp = _sc_gather2(t_src, t_dst, srcp, dstp)
    e1, vals = _edge_update(swi, ndp, e0, bne1, Wne2, bne2, eg, eb,
                            Wne1[D_EMB:], wq_cat, wk_cat)

    zn = jnp.zeros((NROWS_CORE // 16, VW), jnp.float32)
    tabn = _sc_scatter(vals, dstsc, zn)
    num = jnp.concatenate(
        [jnp.concatenate([tabn[p, 0, :NROWS_HALF],
                          tabn[p, 1, :N_NODES - NROWS_HALF]], axis=0)
         for p in range(2)], axis=1)

    node_emb2, t_out = _node_update(num, node_emb, eg, eb, Wen1[:HEADS * D_EMB],
                                    Wen1[HEADS * D_EMB:], ben1, Wen2, ben2,
                                    ng, nb, Wo1[:D_EMB])

    us, ud = _sc_gather2(t_out, t_out, srcp, dstp)
    pred = _out_head(us, ud, e1, Wo1[D_EMB:], bo1, Wo2, bo2)
    return jnp.squeeze(pred[:N_EDGES], axis=-1)


# speedup vs baseline: 1.0378x; 1.0378x over previous
import functools

import jax
import jax.numpy as jnp
from jax import lax
from jax.experimental import pallas as pl
from jax.experimental.pallas import tpu as pltpu
from jax.experimental.pallas import tpu_sc as plsc

N_NODES = 10000
N_EDGES = 320000
D_EMB = 64
HEADS = 4
EPS = 1e-6

NEP = 327680
VW = 128
NROWS_HALF = 5120
NROWS_CORE = 5248
SLOW_CORE = 0
N_SLOW_CHUNKS = 80


def _dot(a, b):
    return jnp.dot(a, b)


def _ln(x, g, b):
    mu = jnp.mean(x, axis=-1, keepdims=True)
    v = jnp.mean((x - mu) * (x - mu), axis=-1, keepdims=True)
    return (x - mu) * jax.lax.rsqrt(v + EPS) * g + b


def _mlp2_body(x, w1, b1, w2, b2, o):
    h = jnp.maximum(_dot(x[...], w1[...]) + b1[...], 0.0)
    o[...] = _dot(h, w2[...]) + b2[...]


def _mlp2(x, w1, b1, w2, b2, blk):
    n = x.shape[0]
    return pl.pallas_call(
        _mlp2_body,
        grid=(n // blk,),
        in_specs=[
            pl.BlockSpec((blk, x.shape[1]), lambda i: (i, 0)),
            pl.BlockSpec(w1.shape, lambda i: (0, 0)),
            pl.BlockSpec(b1.shape, lambda i: (0,)),
            pl.BlockSpec(w2.shape, lambda i: (0, 0)),
            pl.BlockSpec(b2.shape, lambda i: (0,)),
        ],
        out_specs=pl.BlockSpec((blk, w2.shape[1]), lambda i: (i, 0)),
        out_shape=jax.ShapeDtypeStruct((n, w2.shape[1]), jnp.float32),
    )(x, w1, b1, w2, b2)


def _mlp2p_body(x, w1, b1, w2, b2, wp, o, op):
    h = jnp.maximum(_dot(x[...], w1[...]) + b1[...], 0.0)
    e = _dot(h, w2[...]) + b2[...]
    o[...] = e
    op[...] = _dot(e, wp[...])


def _mlp2p(x, w1, b1, w2, b2, wp, blk):
    n = x.shape[0]
    return pl.pallas_call(
        _mlp2p_body,
        grid=(n // blk,),
        in_specs=[
            pl.BlockSpec((blk, x.shape[1]), lambda i: (i, 0)),
            pl.BlockSpec(w1.shape, lambda i: (0, 0)),
            pl.BlockSpec(b1.shape, lambda i: (0,)),
            pl.BlockSpec(w2.shape, lambda i: (0, 0)),
            pl.BlockSpec(b2.shape, lambda i: (0,)),
            pl.BlockSpec(wp.shape, lambda i: (0, 0)),
        ],
        out_specs=[
            pl.BlockSpec((blk, w2.shape[1]), lambda i: (i, 0)),
            pl.BlockSpec((blk, wp.shape[1]), lambda i: (i, 0)),
        ],
        out_shape=[
            jax.ShapeDtypeStruct((n, w2.shape[1]), jnp.float32),
            jax.ShapeDtypeStruct((n, wp.shape[1]), jnp.float32),
        ],
    )(x, w1, b1, w2, b2, wp)


def _edge_update_body(swi, ndp, e0, b1, w2, b2, eg, eb, w1b, wq, wk,
                      o_e1, o_vals):
    h = jnp.maximum(swi[...] + _dot(e0[...], w1b[...]) + b1[...], 0.0)
    upd = _dot(h, w2[...]) + b2[...]
    e1 = _ln(upd + e0[...], eg[...], eb[...])
    o_e1[...] = e1
    kk = _dot(e1, wk[...])
    qq = _dot(ndp[:, :D_EMB], wq[...])
    prod = kk * qq
    idx = lax.broadcasted_iota(jnp.int32, (1, D_EMB), 1)
    e1m = jnp.where(idx < D_EMB - 1, e1, 1.0)
    for c in range(2):
        exa = jnp.exp(jnp.sum(prod[:, (2 * c) * D_EMB:(2 * c + 1) * D_EMB],
                              axis=1, keepdims=True))
        exb = jnp.exp(jnp.sum(prod[:, (2 * c + 1) * D_EMB:(2 * c + 2) * D_EMB],
                              axis=1, keepdims=True))
        o_vals[c] = jnp.concatenate([e1m * exa, e1m * exb], axis=1)


def _edge_update(swi, ndp, e0, b1, w2, b2, eg, eb, w1b, wq, wk):
    blk = 2048
    return pl.pallas_call(
        _edge_update_body,
        grid=(NEP // blk,),
        in_specs=[
            pl.BlockSpec((blk, 2 * D_EMB), lambda i: (i, 0)),
            pl.BlockSpec((blk, 2 * D_EMB), lambda i: (i, 0)),
            pl.BlockSpec((blk, D_EMB), lambda i: (i, 0)),
            pl.BlockSpec(b1.shape, lambda i: (0,)),
            pl.BlockSpec(w2.shape, lambda i: (0, 0)),
            pl.BlockSpec(b2.shape, lambda i: (0,)),
            pl.BlockSpec(eg.shape, lambda i: (0,)),
            pl.BlockSpec(eb.shape, lambda i: (0,)),
            pl.BlockSpec(w1b.shape, lambda i: (0, 0)),
            pl.BlockSpec(wq.shape, lambda i: (0, 0)),
            pl.BlockSpec(wk.shape, lambda i: (0, 0)),
        ],
        out_specs=[
            pl.BlockSpec((blk, D_EMB), lambda i: (i, 0)),
            pl.BlockSpec((2, blk, VW), lambda i: (0, i, 0)),
        ],
        out_shape=[
            jax.ShapeDtypeStruct((NEP, D_EMB), jnp.float32),
            jax.ShapeDtypeStruct((2, NEP, VW), jnp.float32),
        ],
    )(swi, ndp, e0, b1, w2, b2, eg, eb, w1b, wq, wk)


def _node_update_body(num, ne, eg, eb, w1a, w1b, b1, w2, b2, ng, nb, wp, o, op):
    idx = lax.broadcasted_iota(jnp.int32, (1, D_EMB), 1)
    egv, ebv = eg[...], eb[...]
    last = D_EMB - 1
    inv = jnp.where(idx[0] < last, 1.0 / egv, 0.0)
    c2 = jnp.sum(ebv * inv)
    eg63 = jnp.sum(jnp.where(idx[0] == last, egv, 0.0))
    eb63 = jnp.sum(jnp.where(idx[0] == last, ebv, 0.0))
    heads = []
    for h in range(HEADS):
        blk = num[:, h * D_EMB:(h + 1) * D_EMB]
        den = jnp.sum(jnp.where(idx == last, blk, 0.0), axis=1, keepdims=True)
        s1 = jnp.sum(blk * inv, axis=1, keepdims=True)
        n63 = eb63 * den - eg63 * (s1 - den * c2)
        nf = jnp.where(idx < last, blk, n63)
        heads.append(nf / (den + 1e-9))
    attn = jnp.concatenate(heads, axis=1)
    hh = jnp.maximum(_dot(attn, w1a[...]) + _dot(ne[...], w1b[...])
                     + b1[...], 0.0)
    upd = _dot(hh, w2[...]) + b2[...]
    n2 = _ln(upd + ne[...], ng[...], nb[...])
    o[...] = n2
    op[...] = _dot(n2, wp[...])


def _node_update(num, ne, eg, eb, w1a, w1b, b1, w2, b2, ng, nb, wp):
    blk = 2000
    return pl.pallas_call(
        _node_update_body,
        grid=(N_NODES // blk,),
        in_specs=[
            pl.BlockSpec((blk, HEADS * D_EMB), lambda i: (i, 0)),
            pl.BlockSpec((blk, D_EMB), lambda i: (i, 0)),
            pl.BlockSpec(eg.shape, lambda i: (0,)),
            pl.BlockSpec(eb.shape, lambda i: (0,)),
            pl.BlockSpec(w1a.shape, lambda i: (0, 0)),
            pl.BlockSpec(w1b.shape, lambda i: (0, 0)),
            pl.BlockSpec(b1.shape, lambda i: (0,)),
            pl.BlockSpec(w2.shape, lambda i: (0, 0)),
            pl.BlockSpec(b2.shape, lambda i: (0,)),
            pl.BlockSpec(ng.shape, lambda i: (0,)),
            pl.BlockSpec(nb.shape, lambda i: (0,)),
            pl.BlockSpec(wp.shape, lambda i: (0, 0)),
        ],
        out_specs=[
            pl.BlockSpec((blk, D_EMB), lambda i: (i, 0)),
            pl.BlockSpec((blk, wp.shape[1]), lambda i: (i, 0)),
        ],
        out_shape=[
            jax.ShapeDtypeStruct((N_NODES, D_EMB), jnp.float32),
            jax.ShapeDtypeStruct((N_NODES, wp.shape[1]), jnp.float32),
        ],
    )(num, ne, eg, eb, w1a, w1b, b1, w2, b2, ng, nb, wp)


def _out_head_body(us, ud, e1, w1b, b1, w2, b2, o):
    h = jnp.maximum(us[...] + ud[...] + _dot(e1[...], w1b[...]) + b1[...], 0.0)
    o[...] = _dot(h, w2[...]) + b2[...]


def _out_head(us, ud, e1, w1b, b1, w2, b2):
    blk = 2048
    return pl.pallas_call(
        _out_head_body,
        grid=(NEP // blk,),
        in_specs=[
            pl.BlockSpec((blk, 2 * D_EMB), lambda i: (i, 0)),
            pl.BlockSpec((blk, 2 * D_EMB), lambda i: (i, 0)),
            pl.BlockSpec((blk, D_EMB), lambda i: (i, 0)),
            pl.BlockSpec(w1b.shape, lambda i: (0, 0)),
            pl.BlockSpec(b1.shape, lambda i: (0,)),
            pl.BlockSpec(w2.shape, lambda i: (0, 0)),
            pl.BlockSpec(b2.shape, lambda i: (0,)),
        ],
        out_specs=pl.BlockSpec((blk, 1), lambda i: (i, 0)),
        out_shape=jax.ShapeDtypeStruct((NEP, 1), jnp.float32),
    )(us, ud, e1, w1b, b1, w2, b2)


_SC_MESH = dict(core_axis_name="c", subcore_axis_name="s")


def _sc_gather2(ta, tb, ia, ib):
    CH = 128
    PER_W = NEP // 32
    NCH = PER_W // CH
    SUP = 8
    NSUP = NCH // SUP
    W = ta.shape[1]

    @functools.partial(
        pl.kernel, mesh=plsc.VectorSubcoreMesh(**_SC_MESH),
        out_type=[jax.ShapeDtypeStruct((NEP, W), jnp.float32),
                  jax.ShapeDtypeStruct((NEP, W), jnp.float32)],
        scratch_types=[
            pltpu.VMEM((SUP * CH,), jnp.int32),
            pltpu.VMEM((SUP * CH,), jnp.int32),
            pltpu.VMEM((3, CH, W), jnp.float32),
            pltpu.VMEM((3, CH, W), jnp.float32),
            pltpu.SemaphoreType.DMA((3,)),
            pltpu.SemaphoreType.DMA((3,)),
            pltpu.SemaphoreType.DMA((3,)),
            pltpu.SemaphoreType.DMA((3,)),
        ],
    )
    def k(ta_h, tb_h, ia_h, ib_h, oa_h, ob_h, iav, ibv, ra, rb,
          sga, sgb, swa, swb):
        c = lax.axis_index("c")
        s = lax.axis_index("s")
        n_slow = N_SLOW_CHUNKS
        n_fast = 2 * NCH - n_slow
        nsup_me = jnp.where(c == SLOW_CORE, n_slow // SUP, n_fast // SUP)
        base0 = jnp.where(c == SLOW_CORE, s * n_slow,
                          16 * n_slow + s * n_fast) * CH
        NB = 3

        def sup_body(g, carry):
            sbase = base0 + g * (SUP * CH)
            pltpu.sync_copy(ia_h.at[pl.ds(sbase, SUP * CH)], iav)
            pltpu.sync_copy(ib_h.at[pl.ds(sbase, SUP * CH)], ibv)
            for b in range(SUP):
                sl = b % NB
                if b >= NB:
                    pltpu.make_async_copy(
                        ra.at[sl], oa_h.at[pl.ds(sbase, CH)], swa.at[sl]).wait()
                    pltpu.make_async_copy(
                        rb.at[sl], ob_h.at[pl.ds(sbase, CH)], swb.at[sl]).wait()
                pltpu.async_copy(ta_h.at[iav.at[pl.ds(b * CH, CH)]],
                                 ra.at[sl], sga.at[sl])
                pltpu.async_copy(tb_h.at[ibv.at[pl.ds(b * CH, CH)]],
                                 rb.at[sl], sgb.at[sl])
                if b >= 1:
                    psl = (b - 1) % NB
                    pbase = sbase + (b - 1) * CH
                    pltpu.make_async_copy(
                        ta_h.at[iav.at[pl.ds(0, CH)]], ra.at[psl],
                        sga.at[psl]).wait()
                    pltpu.async_copy(ra.at[psl], oa_h.at[pl.ds(pbase, CH)],
                                     swa.at[psl])
                    pltpu.make_async_copy(
                        tb_h.at[ibv.at[pl.ds(0, CH)]], rb.at[psl],
                        sgb.at[psl]).wait()
                    pltpu.async_copy(rb.at[psl], ob_h.at[pl.ds(pbase, CH)],
                                     swb.at[psl])
            lsl = (SUP - 1) % NB
            lbase = sbase + (SUP - 1) * CH
            pltpu.make_async_copy(
                ta_h.at[iav.at[pl.ds(0, CH)]], ra.at[lsl], sga.at[lsl]).wait()
            pltpu.async_copy(ra.at[lsl], oa_h.at[pl.ds(lbase, CH)], swa.at[lsl])
            pltpu.make_async_copy(
                tb_h.at[ibv.at[pl.ds(0, CH)]], rb.at[lsl], sgb.at[lsl]).wait()
            pltpu.async_copy(rb.at[lsl], ob_h.at[pl.ds(lbase, CH)], swb.at[lsl])
            for sl in range(NB):
                pltpu.make_async_copy(
                    ra.at[sl], oa_h.at[pl.ds(sbase, CH)], swa.at[sl]).wait()
                pltpu.make_async_copy(
                    rb.at[sl], ob_h.at[pl.ds(sbase, CH)], swb.at[sl]).wait()
            return carry

        lax.fori_loop(0, nsup_me, sup_body, 0)

    return k(ta, tb, ia, ib)


def _sc_scatter(vals, dstsc, zn):
    CH = 128
    PER_T = NEP // 16
    NCH = PER_T // CH
    SUP = 8
    NSUP = NCH // SUP
    RPT = NROWS_CORE // 16

    @functools.partial(
        pl.kernel, mesh=plsc.VectorSubcoreMesh(**_SC_MESH),
        out_type=jax.ShapeDtypeStruct((2, 2, NROWS_CORE, VW), jnp.float32),
        scratch_types=[
            pltpu.VMEM((SUP, CH), jnp.int32),
            pltpu.VMEM((2, CH, VW), jnp.float32),
            pltpu.VMEM((RPT, VW), jnp.float32),
            pltpu.VMEM_SHARED((NROWS_CORE, VW), jnp.float32),
            pltpu.SemaphoreType.DMA((2,)),
            pltpu.SemaphoreType.DMA((2,)),
        ],
    )
    def k(vals_h, dsts_h, zn_h, on_h, iv, vv, ov, tab, sv, ss):
        c = lax.axis_index("c")
        s = lax.axis_index("s")
        for p in range(2):
            pltpu.sync_copy(zn_h, tab.at[pl.ds(s * RPT, RPT)])
            plsc.subcore_barrier()

            def sup_body(g, carry):
                crow = (s * PER_T + g * (SUP * CH)) // CH
                pltpu.sync_copy(
                    dsts_h.at[c, pl.ds(pl.multiple_of(crow, 8), SUP)], iv)
                for b in range(SUP):
                    sl = b % 2
                    if b >= 2:
                        pltpu.make_async_copy(
                            vv.at[sl], tab.at[iv.at[0]], ss.at[sl]).wait()
                    pltpu.async_copy(
                        vals_h.at[p, pl.ds((crow + b) * CH, CH)],
                        vv.at[sl], sv.at[sl])
                    if b >= 1:
                        psl = (b - 1) % 2
                        pltpu.make_async_copy(
                            vals_h.at[p, pl.ds(crow * CH, CH)],
                            vv.at[psl], sv.at[psl]).wait()
                        pltpu.async_copy(vv.at[psl], tab.at[iv.at[b - 1]],
                                         ss.at[psl], add=True)
                lsl = (SUP - 1) % 2
                pltpu.make_async_copy(
                    vals_h.at[p, pl.ds(crow * CH, CH)],
                    vv.at[lsl], sv.at[lsl]).wait()
                pltpu.async_copy(vv.at[lsl], tab.at[iv.at[SUP - 1]],
                                 ss.at[lsl], add=True)
                for sl in range(2):
                    pltpu.make_async_copy(
                        vv.at[sl], tab.at[iv.at[0]], ss.at[sl]).wait()
                return carry

            lax.fori_loop(0, NSUP, sup_body, 0)
            plsc.subcore_barrier()
            pltpu.sync_copy(tab.at[pl.ds(s * RPT, RPT)], ov)
            pltpu.sync_copy(ov, on_h.at[p, c, pl.ds(s * RPT, RPT)])

    return k(vals, dstsc, zn)


def kernel(node_feats, edge_feats, edge_index, Wn1, bn1, Wn2, bn2, We1, be1, We2, be2,
           Wne1, bne1, Wne2, bne2, Wq, Wk, Wen1, ben1, Wen2, ben2,
           eg, eb, ng, nb, Wo1, bo1, Wo2, bo2):
    src = edge_index[0]
    dst = edge_index[1]
    pad = NEP - N_EDGES
    srcp = jnp.concatenate([src, jnp.zeros((pad,), jnp.int32)])
    dstp = jnp.concatenate([dst, jnp.zeros((pad,), jnp.int32)])
    dsts = jnp.concatenate(
        [dst, N_NODES + (jnp.arange(pad, dtype=jnp.int32) % (2 * NROWS_HALF - N_NODES))])
    dump = NROWS_HALF + (dsts & 127)
    dstsc = jnp.stack([
        jnp.where(dsts < NROWS_HALF, dsts, dump),
        jnp.where(dsts >= NROWS_HALF, dsts - NROWS_HALF, dump),
    ]).reshape(2, NEP // 128, 128)

    scale = 1.0 / jnp.sqrt(jnp.asarray(D_EMB, dtype=jnp.float32))
    wq_cat = Wq.transpose(1, 0, 2).reshape(D_EMB, HEADS * D_EMB) * scale
    wk_cat = Wk.transpose(1, 0, 2).reshape(D_EMB, HEADS * D_EMB)

    node_emb, t_src = _mlp2p(node_feats, Wn1, bn1, Wn2, bn2, Wne1[:D_EMB], 2000)
    t_dst = jnp.pad(node_emb, ((0, 0), (0, D_EMB)))
    swi, ndp = _sc_gather2(t_src, t_dst, srcp, dstp)
    efp = jnp.concatenate([edge_feats,
                           jnp.zeros((pad, edge_feats.shape[1]), jnp.float32)])
    e0 = _mlp2(efp, We1, be1, We2, be2, 2048)
    e1, vals = _edge_update(swi, ndp, e0, bne1, Wne2, bne2, eg, eb,
                            Wne1[D_EMB:], wq_cat, wk_cat)

    zn = jnp.zeros((NROWS_CORE // 16, VW), jnp.float32)
    tabn = _sc_scatter(vals, dstsc, zn)
    num = jnp.concatenate(
        [jnp.concatenate([tabn[p, 0, :NROWS_HALF],
                          tabn[p, 1, :N_NODES - NROWS_HALF]], axis=0)
         for p in range(2)], axis=1)

    node_emb2, t_out = _node_update(num, node_emb, eg, eb, Wen1[:HEADS * D_EMB],
                                    Wen1[HEADS * D_EMB:], ben1, Wen2, ben2,
                                    ng, nb, Wo1[:D_EMB])

    us, ud = _sc_gather2(t_out, t_out, srcp, dstp)
    pred = _out_head(us, ud, e1, Wo1[D_EMB:], bo1, Wo2, bo2)
    return jnp.squeeze(pred[:N_EDGES], axis=-1)


# speedup vs baseline: 1.0483x; 1.0101x over previous
import functools

import jax
import jax.numpy as jnp
from jax import lax
from jax.experimental import pallas as pl
from jax.experimental.pallas import tpu as pltpu
from jax.experimental.pallas import tpu_sc as plsc

N_NODES = 10000
N_EDGES = 320000
D_EMB = 64
HEADS = 4
EPS = 1e-6

NEP = 327680
VW = 128
NROWS_HALF = 5120
NROWS_CORE = 5248
SLOW_CORE = 0
N_SLOW_CHUNKS = 80


def _dot(a, b):
    return jnp.dot(a, b)


def _ln(x, g, b):
    mu = jnp.mean(x, axis=-1, keepdims=True)
    v = jnp.mean((x - mu) * (x - mu), axis=-1, keepdims=True)
    return (x - mu) * jax.lax.rsqrt(v + EPS) * g + b


def _mlp2_body(x, w1, b1, w2, b2, o):
    h = jnp.maximum(_dot(x[...], w1[...]) + b1[...], 0.0)
    o[...] = _dot(h, w2[...]) + b2[...]


def _mlp2(x, w1, b1, w2, b2, blk):
    n = x.shape[0]
    return pl.pallas_call(
        _mlp2_body,
        grid=(n // blk,),
        in_specs=[
            pl.BlockSpec((blk, x.shape[1]), lambda i: (i, 0)),
            pl.BlockSpec(w1.shape, lambda i: (0, 0)),
            pl.BlockSpec(b1.shape, lambda i: (0,)),
            pl.BlockSpec(w2.shape, lambda i: (0, 0)),
            pl.BlockSpec(b2.shape, lambda i: (0,)),
        ],
        out_specs=pl.BlockSpec((blk, w2.shape[1]), lambda i: (i, 0)),
        out_shape=jax.ShapeDtypeStruct((n, w2.shape[1]), jnp.float32),
    )(x, w1, b1, w2, b2)


def _mlp2p_body(x, w1, b1, w2, b2, wp, o, op):
    h = jnp.maximum(_dot(x[...], w1[...]) + b1[...], 0.0)
    e = _dot(h, w2[...]) + b2[...]
    o[...] = e
    op[...] = _dot(e, wp[...])


def _mlp2p(x, w1, b1, w2, b2, wp, blk):
    n = x.shape[0]
    return pl.pallas_call(
        _mlp2p_body,
        grid=(n // blk,),
        in_specs=[
            pl.BlockSpec((blk, x.shape[1]), lambda i: (i, 0)),
            pl.BlockSpec(w1.shape, lambda i: (0, 0)),
            pl.BlockSpec(b1.shape, lambda i: (0,)),
            pl.BlockSpec(w2.shape, lambda i: (0, 0)),
            pl.BlockSpec(b2.shape, lambda i: (0,)),
            pl.BlockSpec(wp.shape, lambda i: (0, 0)),
        ],
        out_specs=[
            pl.BlockSpec((blk, w2.shape[1]), lambda i: (i, 0)),
            pl.BlockSpec((blk, wp.shape[1]), lambda i: (i, 0)),
        ],
        out_shape=[
            jax.ShapeDtypeStruct((n, w2.shape[1]), jnp.float32),
            jax.ShapeDtypeStruct((n, wp.shape[1]), jnp.float32),
        ],
    )(x, w1, b1, w2, b2, wp)


def _edge_update_body(swi, ndp, e0, b1, w2, b2, eg, eb, w1b, wq, wk,
                      o_e1, o_vals):
    h = jnp.maximum(swi[...] + _dot(e0[...], w1b[...]) + b1[...], 0.0)
    upd = _dot(h, w2[...]) + b2[...]
    e1 = _ln(upd + e0[...], eg[...], eb[...])
    o_e1[...] = e1
    kk = _dot(e1, wk[...])
    qq = _dot(ndp[:, :D_EMB], wq[...])
    prod = kk * qq
    idx = lax.broadcasted_iota(jnp.int32, (1, D_EMB), 1)
    e1m = jnp.where(idx < D_EMB - 1, e1, 1.0)
    for c in range(2):
        exa = jnp.exp(jnp.sum(prod[:, (2 * c) * D_EMB:(2 * c + 1) * D_EMB],
                              axis=1, keepdims=True))
        exb = jnp.exp(jnp.sum(prod[:, (2 * c + 1) * D_EMB:(2 * c + 2) * D_EMB],
                              axis=1, keepdims=True))
        o_vals[c] = jnp.concatenate([e1m * exa, e1m * exb], axis=1)


def _edge_update(swi, ndp, e0, b1, w2, b2, eg, eb, w1b, wq, wk):
    blk = 2048
    return pl.pallas_call(
        _edge_update_body,
        grid=(NEP // blk,),
        in_specs=[
            pl.BlockSpec((blk, 2 * D_EMB), lambda i: (i, 0)),
            pl.BlockSpec((blk, 2 * D_EMB), lambda i: (i, 0)),
            pl.BlockSpec((blk, D_EMB), lambda i: (i, 0)),
            pl.BlockSpec(b1.shape, lambda i: (0,)),
            pl.BlockSpec(w2.shape, lambda i: (0, 0)),
            pl.BlockSpec(b2.shape, lambda i: (0,)),
            pl.BlockSpec(eg.shape, lambda i: (0,)),
            pl.BlockSpec(eb.shape, lambda i: (0,)),
            pl.BlockSpec(w1b.shape, lambda i: (0, 0)),
            pl.BlockSpec(wq.shape, lambda i: (0, 0)),
            pl.BlockSpec(wk.shape, lambda i: (0, 0)),
        ],
        out_specs=[
            pl.BlockSpec((blk, D_EMB), lambda i: (i, 0)),
            pl.BlockSpec((2, blk, VW), lambda i: (0, i, 0)),
        ],
        out_shape=[
            jax.ShapeDtypeStruct((NEP, D_EMB), jnp.float32),
            jax.ShapeDtypeStruct((2, NEP, VW), jnp.float32),
        ],
    )(swi, ndp, e0, b1, w2, b2, eg, eb, w1b, wq, wk)


def _node_update_body(num, ne, eg, eb, w1a, w1b, b1, w2, b2, ng, nb, wp, o, op):
    idx = lax.broadcasted_iota(jnp.int32, (1, D_EMB), 1)
    egv, ebv = eg[...], eb[...]
    last = D_EMB - 1
    inv = jnp.where(idx[0] < last, 1.0 / egv, 0.0)
    c2 = jnp.sum(ebv * inv)
    eg63 = jnp.sum(jnp.where(idx[0] == last, egv, 0.0))
    eb63 = jnp.sum(jnp.where(idx[0] == last, ebv, 0.0))
    heads = []
    for h in range(HEADS):
        blk = num[:, h * D_EMB:(h + 1) * D_EMB]
        den = jnp.sum(jnp.where(idx == last, blk, 0.0), axis=1, keepdims=True)
        s1 = jnp.sum(blk * inv, axis=1, keepdims=True)
        n63 = eb63 * den - eg63 * (s1 - den * c2)
        nf = jnp.where(idx < last, blk, n63)
        heads.append(nf / (den + 1e-9))
    attn = jnp.concatenate(heads, axis=1)
    hh = jnp.maximum(_dot(attn, w1a[...]) + _dot(ne[...], w1b[...])
                     + b1[...], 0.0)
    upd = _dot(hh, w2[...]) + b2[...]
    n2 = _ln(upd + ne[...], ng[...], nb[...])
    o[...] = n2
    op[...] = _dot(n2, wp[...])


def _node_update(num, ne, eg, eb, w1a, w1b, b1, w2, b2, ng, nb, wp):
    blk = 2000
    return pl.pallas_call(
        _node_update_body,
        grid=(N_NODES // blk,),
        in_specs=[
            pl.BlockSpec((blk, HEADS * D_EMB), lambda i: (i, 0)),
            pl.BlockSpec((blk, D_EMB), lambda i: (i, 0)),
            pl.BlockSpec(eg.shape, lambda i: (0,)),
            pl.BlockSpec(eb.shape, lambda i: (0,)),
            pl.BlockSpec(w1a.shape, lambda i: (0, 0)),
            pl.BlockSpec(w1b.shape, lambda i: (0, 0)),
            pl.BlockSpec(b1.shape, lambda i: (0,)),
            pl.BlockSpec(w2.shape, lambda i: (0, 0)),
            pl.BlockSpec(b2.shape, lambda i: (0,)),
            pl.BlockSpec(ng.shape, lambda i: (0,)),
            pl.BlockSpec(nb.shape, lambda i: (0,)),
            pl.BlockSpec(wp.shape, lambda i: (0, 0)),
        ],
        out_specs=[
            pl.BlockSpec((blk, D_EMB), lambda i: (i, 0)),
            pl.BlockSpec((blk, wp.shape[1]), lambda i: (i, 0)),
        ],
        out_shape=[
            jax.ShapeDtypeStruct((N_NODES, D_EMB), jnp.float32),
            jax.ShapeDtypeStruct((N_NODES, wp.shape[1]), jnp.float32),
        ],
    )(num, ne, eg, eb, w1a, w1b, b1, w2, b2, ng, nb, wp)


def _out_head_body(us, ud, e1, w1b, b1, w2, b2, o):
    h = jnp.maximum(us[...] + ud[...] + _dot(e1[...], w1b[...]) + b1[...], 0.0)
    o[...] = _dot(h, w2[...]) + b2[...]


def _out_head(us, ud, e1, w1b, b1, w2, b2):
    blk = 2048
    return pl.pallas_call(
        _out_head_body,
        grid=(NEP // blk,),
        in_specs=[
            pl.BlockSpec((blk, 2 * D_EMB), lambda i: (i, 0)),
            pl.BlockSpec((blk, 2 * D_EMB), lambda i: (i, 0)),
            pl.BlockSpec((blk, D_EMB), lambda i: (i, 0)),
            pl.BlockSpec(w1b.shape, lambda i: (0, 0)),
            pl.BlockSpec(b1.shape, lambda i: (0,)),
            pl.BlockSpec(w2.shape, lambda i: (0, 0)),
            pl.BlockSpec(b2.shape, lambda i: (0,)),
        ],
        out_specs=pl.BlockSpec((blk, 1), lambda i: (i, 0)),
        out_shape=jax.ShapeDtypeStruct((NEP, 1), jnp.float32),
    )(us, ud, e1, w1b, b1, w2, b2)


_SC_MESH = dict(core_axis_name="c", subcore_axis_name="s")


def _sc_gather2(ta, tb, ia, ib):
    CH = 128
    PER_W = NEP // 32
    NCH = PER_W // CH
    SUP = 16
    NSUP = NCH // SUP
    W = ta.shape[1]

    @functools.partial(
        pl.kernel, mesh=plsc.VectorSubcoreMesh(**_SC_MESH),
        out_type=[jax.ShapeDtypeStruct((NEP, W), jnp.float32),
                  jax.ShapeDtypeStruct((NEP, W), jnp.float32)],
        scratch_types=[
            pltpu.VMEM((SUP * CH,), jnp.int32),
            pltpu.VMEM((SUP * CH,), jnp.int32),
            pltpu.VMEM((3, CH, W), jnp.float32),
            pltpu.VMEM((3, CH, W), jnp.float32),
            pltpu.SemaphoreType.DMA((3,)),
            pltpu.SemaphoreType.DMA((3,)),
            pltpu.SemaphoreType.DMA((3,)),
            pltpu.SemaphoreType.DMA((3,)),
        ],
    )
    def k(ta_h, tb_h, ia_h, ib_h, oa_h, ob_h, iav, ibv, ra, rb,
          sga, sgb, swa, swb):
        c = lax.axis_index("c")
        s = lax.axis_index("s")
        n_slow = N_SLOW_CHUNKS
        n_fast = 2 * NCH - n_slow
        nsup_me = jnp.where(c == SLOW_CORE, n_slow // SUP, n_fast // SUP)
        base0 = jnp.where(c == SLOW_CORE, s * n_slow,
                          16 * n_slow + s * n_fast) * CH
        NB = 3

        def sup_body(g, carry):
            sbase = base0 + g * (SUP * CH)
            pltpu.sync_copy(ia_h.at[pl.ds(sbase, SUP * CH)], iav)
            pltpu.sync_copy(ib_h.at[pl.ds(sbase, SUP * CH)], ibv)
            for b in range(SUP):
                sl = b % NB
                if b >= NB:
                    pltpu.make_async_copy(
                        ra.at[sl], oa_h.at[pl.ds(sbase, CH)], swa.at[sl]).wait()
                    pltpu.make_async_copy(
                        rb.at[sl], ob_h.at[pl.ds(sbase, CH)], swb.at[sl]).wait()
                pltpu.async_copy(ta_h.at[iav.at[pl.ds(b * CH, CH)]],
                                 ra.at[sl], sga.at[sl])
                pltpu.async_copy(tb_h.at[ibv.at[pl.ds(b * CH, CH)]],
                                 rb.at[sl], sgb.at[sl])
                if b >= 1:
                    psl = (b - 1) % NB
                    pbase = sbase + (b - 1) * CH
                    pltpu.make_async_copy(
                        ta_h.at[iav.at[pl.ds(0, CH)]], ra.at[psl],
                        sga.at[psl]).wait()
                    pltpu.async_copy(ra.at[psl], oa_h.at[pl.ds(pbase, CH)],
                                     swa.at[psl])
                    pltpu.make_async_copy(
                        tb_h.at[ibv.at[pl.ds(0, CH)]], rb.at[psl],
                        sgb.at[psl]).wait()
                    pltpu.async_copy(rb.at[psl], ob_h.at[pl.ds(pbase, CH)],
                                     swb.at[psl])
            lsl = (SUP - 1) % NB
            lbase = sbase + (SUP - 1) * CH
            pltpu.make_async_copy(
                ta_h.at[iav.at[pl.ds(0, CH)]], ra.at[lsl], sga.at[lsl]).wait()
            pltpu.async_copy(ra.at[lsl], oa_h.at[pl.ds(lbase, CH)], swa.at[lsl])
            pltpu.make_async_copy(
                tb_h.at[ibv.at[pl.ds(0, CH)]], rb.at[lsl], sgb.at[lsl]).wait()
            pltpu.async_copy(rb.at[lsl], ob_h.at[pl.ds(lbase, CH)], swb.at[lsl])
            for sl in range(NB):
                pltpu.make_async_copy(
                    ra.at[sl], oa_h.at[pl.ds(sbase, CH)], swa.at[sl]).wait()
                pltpu.make_async_copy(
                    rb.at[sl], ob_h.at[pl.ds(sbase, CH)], swb.at[sl]).wait()
            return carry

        lax.fori_loop(0, nsup_me, sup_body, 0)

    return k(ta, tb, ia, ib)


def _sc_scatter(vals, dstsc, zn):
    CH = 128
    PER_T = NEP // 16
    NCH = PER_T // CH
    SUP = 16
    NSUP = NCH // SUP
    RPT = NROWS_CORE // 16

    @functools.partial(
        pl.kernel, mesh=plsc.VectorSubcoreMesh(**_SC_MESH),
        out_type=jax.ShapeDtypeStruct((2, 2, NROWS_CORE, VW), jnp.float32),
        scratch_types=[
            pltpu.VMEM((SUP, CH), jnp.int32),
            pltpu.VMEM((2, CH, VW), jnp.float32),
            pltpu.VMEM((RPT, VW), jnp.float32),
            pltpu.VMEM_SHARED((NROWS_CORE, VW), jnp.float32),
            pltpu.SemaphoreType.DMA((2,)),
            pltpu.SemaphoreType.DMA((2,)),
        ],
    )
    def k(vals_h, dsts_h, zn_h, on_h, iv, vv, ov, tab, sv, ss):
        c = lax.axis_index("c")
        s = lax.axis_index("s")
        for p in range(2):
            pltpu.sync_copy(zn_h, tab.at[pl.ds(s * RPT, RPT)])
            plsc.subcore_barrier()

            def sup_body(g, carry):
                crow = (s * PER_T + g * (SUP * CH)) // CH
                pltpu.sync_copy(
                    dsts_h.at[c, pl.ds(pl.multiple_of(crow, 8), SUP)], iv)
                for b in range(SUP):
                    sl = b % 2
                    if b >= 2:
                        pltpu.make_async_copy(
                            vv.at[sl], tab.at[iv.at[0]], ss.at[sl]).wait()
                    pltpu.async_copy(
                        vals_h.at[p, pl.ds((crow + b) * CH, CH)],
                        vv.at[sl], sv.at[sl])
                    if b >= 1:
                        psl = (b - 1) % 2
                        pltpu.make_async_copy(
                            vals_h.at[p, pl.ds(crow * CH, CH)],
                            vv.at[psl], sv.at[psl]).wait()
                        pltpu.async_copy(vv.at[psl], tab.at[iv.at[b - 1]],
                                         ss.at[psl], add=True)
                lsl = (SUP - 1) % 2
                pltpu.make_async_copy(
                    vals_h.at[p, pl.ds(crow * CH, CH)],
                    vv.at[lsl], sv.at[lsl]).wait()
                pltpu.async_copy(vv.at[lsl], tab.at[iv.at[SUP - 1]],
                                 ss.at[lsl], add=True)
                for sl in range(2):
                    pltpu.make_async_copy(
                        vv.at[sl], tab.at[iv.at[0]], ss.at[sl]).wait()
                return carry

            lax.fori_loop(0, NSUP, sup_body, 0)
            plsc.subcore_barrier()
            pltpu.sync_copy(tab.at[pl.ds(s * RPT, RPT)], ov)
            pltpu.sync_copy(ov, on_h.at[p, c, pl.ds(s * RPT, RPT)])

    return k(vals, dstsc, zn)


def kernel(node_feats, edge_feats, edge_index, Wn1, bn1, Wn2, bn2, We1, be1, We2, be2,
           Wne1, bne1, Wne2, bne2, Wq, Wk, Wen1, ben1, Wen2, ben2,
           eg, eb, ng, nb, Wo1, bo1, Wo2, bo2):
    src = edge_index[0]
    dst = edge_index[1]
    pad = NEP - N_EDGES
    srcp = jnp.concatenate([src, jnp.zeros((pad,), jnp.int32)])
    dstp = jnp.concatenate([dst, jnp.zeros((pad,), jnp.int32)])
    dsts = jnp.concatenate(
        [dst, N_NODES + (jnp.arange(pad, dtype=jnp.int32) % (2 * NROWS_HALF - N_NODES))])
    dump = NROWS_HALF + (dsts & 127)
    dstsc = jnp.stack([
        jnp.where(dsts < NROWS_HALF, dsts, dump),
        jnp.where(dsts >= NROWS_HALF, dsts - NROWS_HALF, dump),
    ]).reshape(2, NEP // 128, 128)

    scale = 1.0 / jnp.sqrt(jnp.asarray(D_EMB, dtype=jnp.float32))
    wq_cat = Wq.transpose(1, 0, 2).reshape(D_EMB, HEADS * D_EMB) * scale
    wk_cat = Wk.transpose(1, 0, 2).reshape(D_EMB, HEADS * D_EMB)

    node_emb, t_src = _mlp2p(node_feats, Wn1, bn1, Wn2, bn2, Wne1[:D_EMB], 2000)
    t_dst = jnp.pad(node_emb, ((0, 0), (0, D_EMB)))
    swi, ndp = _sc_gather2(t_src, t_dst, srcp, dstp)
    efp = jnp.concatenate([edge_feats,
                           jnp.zeros((pad, edge_feats.shape[1]), jnp.float32)])
    e0 = _mlp2(efp, We1, be1, We2, be2, 2048)
    e1, vals = _edge_update(swi, ndp, e0, bne1, Wne2, bne2, eg, eb,
                            Wne1[D_EMB:], wq_cat, wk_cat)

    zn = jnp.zeros((NROWS_CORE // 16, VW), jnp.float32)
    tabn = _sc_scatter(vals, dstsc, zn)
    num = jnp.concatenate(
        [jnp.concatenate([tabn[p, 0, :NROWS_HALF],
                          tabn[p, 1, :N_NODES - NROWS_HALF]], axis=0)
         for p in range(2)], axis=1)

    node_emb2, t_out = _node_update(num, node_emb, eg, eb, Wen1[:HEADS * D_EMB],
                                    Wen1[HEADS * D_EMB:], ben1, Wen2, ben2,
                                    ng, nb, Wo1[:D_EMB])

    us, ud = _sc_gather2(t_out, t_out, srcp, dstp)
    pred = _out_head(us, ud, e1, Wo1[D_EMB:], bo1, Wo2, bo2)
    return jnp.squeeze(pred[:N_EDGES], axis=-1)
